# baseline (speedup 1.0000x reference)
import jax
import jax.numpy as jnp
from jax import lax
from jax.experimental import pallas as pl
from jax.experimental.pallas import tpu as pltpu
from jax.experimental.pallas import tpu_sc as plsc

DEPTH = 1000
N = 4096
J = 26
NUM_CORES = 2
NUM_SUBCORES = 16
NW = NUM_CORES * NUM_SUBCORES
TROWS = DEPTH // 8
NSLAB = J * TROWS
BASE = NSLAB // NW
EXTRA = NSLAB - BASE * NW
NBUF = 2
LANE = 16
NITER = (BASE + 1 + NBUF - 1) // NBUF


def _onehot_sc_body(idx_hbm, onoff_hbm, out_hbm,
                    idxrow_v, onoff_v, buf0, buf1, sem0, sem1):
    wid = lax.axis_index("s") * NUM_CORES + lax.axis_index("c")
    nslab = jnp.where(wid < EXTRA, BASE + 1, BASE)
    s0 = jnp.where(wid < EXTRA, wid * (BASE + 1),
                   EXTRA * (BASE + 1) + (wid - EXTRA) * BASE)
    j0 = s0 // TROWS

    pltpu.sync_copy(idx_hbm.at[pl.ds(j0 * N, 2 * N)], idxrow_v)
    pltpu.sync_copy(onoff_hbm, onoff_v)
    on_vec = onoff_v[pl.ds(0, LANE)]
    off_vec = onoff_v[pl.ds(LANE, LANE)]
    lane = lax.iota(jnp.int32, LANE)

    bufs = (buf0, buf1)
    sems = (sem0, sem1)

    def fill(buf):
        for r in range(8):
            def body(k, c):
                buf[r, pl.ds(k * LANE, LANE)] = off_vec
                return c
            lax.fori_loop(0, N // LANE, body, 0)

    fill(buf0)
    fill(buf1)

    def slab_jt(s):
        j = s // TROWS
        return j, s - j * TROWS

    def scat_slab(buf, s, val):
        j, t = slab_jt(s)
        jj = j - j0
        d0 = t * 8

        def body(k, c):
            d16 = idxrow_v[pl.ds(jj * N + k * LANE, LANE)]
            rel = d16 - d0
            m = (rel >= 0) & (rel < 8)
            i16 = k * LANE + lane
            plsc.store_scatter(buf, [rel, i16], val, mask=m)
            return c
        lax.fori_loop(0, N // LANE, body, 0)

    def start_dma(buf, sem, s):
        j, t = slab_jt(s)
        d0 = pl.multiple_of(t * 8, 8)
        pltpu.async_copy(buf, out_hbm.at[j, pl.ds(d0, 8)], sem)

    def wait_dma(buf, sem, s):
        j, t = slab_jt(s)
        d0 = pl.multiple_of(t * 8, 8)
        pltpu.make_async_copy(buf, out_hbm.at[j, pl.ds(d0, 8)], sem).wait()

    def step(i, c):
        for b in range(NBUF):
            g = i * NBUF + b

            @pl.when(g < nslab)
            def _():
                @pl.when(g >= NBUF)
                def _():
                    wait_dma(bufs[b], sems[b], s0 + g - NBUF)
                    scat_slab(bufs[b], s0 + g - NBUF, off_vec)

                scat_slab(bufs[b], s0 + g, on_vec)
                start_dma(bufs[b], sems[b], s0 + g)
        return c

    lax.fori_loop(0, NITER, step, 0)

    for b in range(NBUF):
        last_g = nslab - 1 - ((nslab - 1 - b) % NBUF)
        wait_dma(bufs[b], sems[b], s0 + last_g)


def kernel(indices, on_value, off_value):
    idx_t = indices.T.astype(jnp.int32)
    idx_t = jnp.pad(idx_t, ((0, 1), (0, 0))).reshape(-1)
    onoff = jnp.concatenate([
        jnp.full((LANE,), on_value, jnp.float32),
        jnp.full((LANE,), off_value, jnp.float32),
    ])
    mesh = plsc.VectorSubcoreMesh(
        core_axis_name="c", subcore_axis_name="s",
        num_cores=NUM_CORES, num_subcores=NUM_SUBCORES)
    out = pl.kernel(
        _onehot_sc_body,
        out_type=jax.ShapeDtypeStruct((J, DEPTH, N), jnp.float32),
        mesh=mesh,
        compiler_params=pltpu.CompilerParams(
            needs_layout_passes=False, use_tc_tiling_on_sc=True),
        scratch_types=(
            [pltpu.VMEM((2 * N,), jnp.int32),
             pltpu.VMEM((2 * LANE,), jnp.float32),
             pltpu.VMEM((8, N), jnp.float32),
             pltpu.VMEM((8, N), jnp.float32)]
            + [pltpu.SemaphoreType.DMA] * NBUF
        ),
    )(idx_t, onoff)
    return jnp.transpose(out, (2, 0, 1))
